# Initial kernel scaffold; baseline (speedup 1.0000x reference)
import functools

import jax
import jax.numpy as jnp
from jax import lax
from jax.experimental import pallas as pl
from jax.experimental.pallas import tpu as pltpu, tpu_sc as plsc

N = 10000
E = 320000
D = 128
H1 = 16

NC = 2
NS = 16
NW = NC * NS
EPW = E // NW
CH = 80
NCHUNK = EPW // CH
RPT = N // NS


def _agg_body(p_hbm, src_hbm, dst_hbm, w_hbm, out_hbm,
              src_v, dst_v, w_v, rows_v, stage_v, acc_sh, sem):
    c = lax.axis_index("c")
    s = lax.axis_index("s")
    wid = c * NS + s

    pltpu.sync_copy(src_hbm.at[wid], src_v)
    pltpu.sync_copy(dst_hbm.at[wid], dst_v)
    pltpu.sync_copy(w_hbm.at[wid], w_v)

    def _wplus(i, carry):
        w_v[pl.ds(i * 16, 16)] = w_v[pl.ds(i * 16, 16)] + 1.0
        return carry
    lax.fori_loop(0, EPW // 16, _wplus, 0)

    def _zero(i, carry):
        stage_v[i] = jnp.zeros((16,), jnp.float32)
        return carry
    lax.fori_loop(0, RPT, _zero, 0)
    pltpu.sync_copy(stage_v, acc_sh.at[pl.ds(s * RPT, RPT)])
    plsc.subcore_barrier()

    def _chunk(j, carry):
        pltpu.async_copy(p_hbm.at[src_v.at[j]], rows_v, sem).wait()

        def _edge(r, carry2):
            e = j * CH + r
            wspl = plsc.load_gather(w_v, [jnp.full((16,), e, jnp.int32)])
            rows_v[r] = rows_v[r] * wspl
            return carry2
        lax.fori_loop(0, CH, _edge, 0)

        pltpu.sync_copy(rows_v, acc_sh.at[dst_v.at[j]], add=True)
        return carry
    lax.fori_loop(0, NCHUNK, _chunk, 0)
    plsc.subcore_barrier()

    pltpu.sync_copy(acc_sh.at[pl.ds(s * RPT, RPT)], stage_v)
    pltpu.sync_copy(stage_v, out_hbm.at[c, pl.ds(s * RPT, RPT)])


_agg = functools.partial(
    pl.kernel,
    out_type=jax.ShapeDtypeStruct((NC, N, H1), jnp.float32),
    mesh=plsc.VectorSubcoreMesh(core_axis_name="c", subcore_axis_name="s"),
    scratch_types=[
        pltpu.VMEM((NCHUNK, CH), jnp.int32),
        pltpu.VMEM((NCHUNK, CH), jnp.int32),
        pltpu.VMEM((EPW,), jnp.float32),
        pltpu.VMEM((CH, H1), jnp.float32),
        pltpu.VMEM((RPT, H1), jnp.float32),
        pltpu.VMEM_SHARED((N, H1), jnp.float32),
        pltpu.SemaphoreType.DMA,
    ],
)(_agg_body)


def _proj_body(x_ref, wt_ref, o_ref):
    o_ref[...] = jnp.dot(x_ref[...], wt_ref[...],
                         preferred_element_type=jnp.float32)


def _combine_relu_body(p_ref, parts_ref, sw_ref, b_ref, o_ref):
    acc = parts_ref[0] + parts_ref[1]
    o_ref[...] = jnp.maximum(
        p_ref[...] * sw_ref[...] + acc + b_ref[...], 0.0)


def _combine_mm_body(x_ref, parts_ref, sw_ref, wt_ref, b_ref, o_ref):
    h2 = x_ref[...] * sw_ref[...] + parts_ref[0] + parts_ref[1]
    o_ref[...] = jnp.dot(h2, wt_ref[...],
                         preferred_element_type=jnp.float32) + b_ref[...]


def kernel(features, edge_index, edge_weight, self_weight, W1, b1, W2, b2):
    src = edge_index[0].reshape(NW, NCHUNK, CH)
    dst = edge_index[1].reshape(NW, NCHUNK, CH)
    w = edge_weight.reshape(NW, EPW)
    swp1 = self_weight + 1.0

    p1 = pl.pallas_call(
        _proj_body,
        out_shape=jax.ShapeDtypeStruct((N, H1), jnp.float32),
    )(features, W1.T)

    parts1 = _agg(p1, src, dst, w)

    x = pl.pallas_call(
        _combine_relu_body,
        out_shape=jax.ShapeDtypeStruct((N, H1), jnp.float32),
    )(p1, parts1, swp1, b1.reshape(1, H1))

    parts2 = _agg(x, src, dst, w)

    out = pl.pallas_call(
        _combine_mm_body,
        out_shape=jax.ShapeDtypeStruct((N, W2.shape[0]), jnp.float32),
    )(x, parts2, swp1, W2.T, b2.reshape(1, W2.shape[0]))

    return out


# speedup vs baseline: 9.8431x; 9.8431x over previous
import functools

import jax
import jax.numpy as jnp
from jax import lax
from jax.experimental import pallas as pl
from jax.experimental.pallas import tpu as pltpu, tpu_sc as plsc

N = 10000
E = 320000
D = 128
H1 = 16

NC = 2
NS = 16
NW = NC * NS
EPW = E // NW
CH = 80
NCHUNK = EPW // CH
NPAD = 10240
RPT = NPAD // NS


def _agg_body(p_hbm, src_hbm, dst_hbm, w_hbm, out_hbm,
              src_v, dst_v, w_v, rows_v, stage_v, acc_sh, sem):
    c = lax.axis_index("c")
    s = lax.axis_index("s")
    wid = c * NS + s

    pltpu.sync_copy(src_hbm.at[wid], src_v)
    pltpu.sync_copy(dst_hbm.at[wid], dst_v)
    pltpu.sync_copy(w_hbm.at[wid], w_v)

    def _wplus(i, carry):
        w_v[pl.ds(i * 16, 16)] = w_v[pl.ds(i * 16, 16)] + 1.0
        return carry
    lax.fori_loop(0, EPW // 16, _wplus, 0)

    def _zero(i, carry):
        stage_v[i] = jnp.zeros((16,), jnp.float32)
        return carry
    lax.fori_loop(0, RPT, _zero, 0)
    pltpu.sync_copy(stage_v, acc_sh.at[pl.ds(s * RPT, RPT)])
    plsc.subcore_barrier()

    def _chunk(j, carry):
        pltpu.async_copy(p_hbm.at[src_v.at[j]], rows_v, sem).wait()

        def _grp(g, carry2):
            wv16 = w_v[pl.ds(pl.multiple_of(j * CH + g * 16, 16), 16)]
            base = g * 16
            for r in range(16):
                rows_v[base + r] = rows_v[base + r] * wv16[r]
            return carry2
        lax.fori_loop(0, CH // 16, _grp, 0)

        pltpu.sync_copy(rows_v, acc_sh.at[dst_v.at[j]], add=True)
        return carry
    lax.fori_loop(0, NCHUNK, _chunk, 0)
    plsc.subcore_barrier()

    pltpu.sync_copy(acc_sh.at[pl.ds(s * RPT, RPT)], stage_v)
    pltpu.sync_copy(stage_v, out_hbm.at[c, pl.ds(s * RPT, RPT)])


_agg = functools.partial(
    pl.kernel,
    out_type=jax.ShapeDtypeStruct((NC, NPAD, H1), jnp.float32),
    mesh=plsc.VectorSubcoreMesh(core_axis_name="c", subcore_axis_name="s"),
    compiler_params=pltpu.CompilerParams(use_tc_tiling_on_sc=False),
    scratch_types=[
        pltpu.VMEM((NCHUNK, CH), jnp.int32),
        pltpu.VMEM((NCHUNK, CH), jnp.int32),
        pltpu.VMEM((EPW,), jnp.float32),
        pltpu.VMEM((CH, H1), jnp.float32),
        pltpu.VMEM((RPT, H1), jnp.float32),
        pltpu.VMEM_SHARED((NPAD, H1), jnp.float32),
        pltpu.SemaphoreType.DMA,
    ],
)(_agg_body)


def _proj_body(x_ref, wt_ref, o_ref):
    o_ref[...] = jnp.dot(x_ref[...], wt_ref[...],
                         preferred_element_type=jnp.float32,
                         precision=jax.lax.Precision.HIGHEST)


def _combine_relu_body(p_ref, parts_ref, sw_ref, b_ref, o_ref):
    acc = parts_ref[0] + parts_ref[1]
    o_ref[...] = jnp.maximum(
        p_ref[...] * sw_ref[...] + acc + b_ref[...], 0.0)


def _combine_mm_body(x_ref, parts_ref, sw_ref, wt_ref, b_ref, o_ref):
    h2 = x_ref[...] * sw_ref[...] + parts_ref[0] + parts_ref[1]
    o_ref[...] = jnp.dot(h2, wt_ref[...],
                         preferred_element_type=jnp.float32,
                         precision=jax.lax.Precision.HIGHEST) + b_ref[...]


def kernel(features, edge_index, edge_weight, self_weight, W1, b1, W2, b2):
    src = edge_index[0].reshape(NW, NCHUNK, CH)
    dst = edge_index[1].reshape(NW, NCHUNK, CH)
    w = edge_weight.reshape(NW, EPW)
    swp1 = self_weight + 1.0

    p1 = pl.pallas_call(
        _proj_body,
        out_shape=jax.ShapeDtypeStruct((N, H1), jnp.float32),
    )(features, W1.T)

    parts1 = _agg(p1, src, dst, w)[:, :N, :]

    x = pl.pallas_call(
        _combine_relu_body,
        out_shape=jax.ShapeDtypeStruct((N, H1), jnp.float32),
    )(p1, parts1, swp1, b1.reshape(1, H1))

    parts2 = _agg(x, src, dst, w)[:, :N, :]

    out = pl.pallas_call(
        _combine_mm_body,
        out_shape=jax.ShapeDtypeStruct((N, W2.shape[0]), jnp.float32),
    )(x, parts2, swp1, W2.T, b2.reshape(1, W2.shape[0]))

    return out


# speedup vs baseline: 13.4467x; 1.3661x over previous
import functools

import jax
import jax.numpy as jnp
from jax import lax
from jax.experimental import pallas as pl
from jax.experimental.pallas import tpu as pltpu, tpu_sc as plsc

N = 10000
E = 320000
D = 128
H1 = 16

NC = 2
NS = 16
NW = NC * NS
EPW = E // NW
CH = 128
EPWP = 10240
NCHUNK = EPWP // CH
NBUF = 4
NSTEP = NCHUNK // NBUF
NPAD = 10240
RPT = NPAD // NS
PAD_DST = NPAD - 8


def _agg_body(p_hbm, src_hbm, dst_hbm, w_hbm, out_hbm,
              src_v, dst_v, w_v, gbuf, sbuf, stage_v, acc_sh, gsem, ssem):
    c = lax.axis_index("c")
    s = lax.axis_index("s")
    wid = c * NS + s

    pltpu.sync_copy(src_hbm.at[wid], src_v)
    pltpu.sync_copy(dst_hbm.at[wid], dst_v)
    pltpu.sync_copy(w_hbm.at[wid], w_v)

    def _wplus(i, carry):
        w_v[pl.ds(i * 16, 16)] = w_v[pl.ds(i * 16, 16)] + 1.0
        return carry
    lax.fori_loop(0, EPWP // 16, _wplus, 0)

    def _zero(i, carry):
        stage_v[i] = jnp.zeros((16,), jnp.float32)
        return carry
    lax.fori_loop(0, RPT, _zero, 0)
    pltpu.sync_copy(stage_v, acc_sh.at[pl.ds(s * RPT, RPT)])
    plsc.subcore_barrier()

    def _gwait(b):
        pltpu.make_async_copy(p_hbm.at[pl.ds(0, CH)], gbuf.at[b],
                              gsem.at[b]).wait()

    def _swait(b):
        pltpu.make_async_copy(p_hbm.at[pl.ds(0, CH)], sbuf.at[b],
                              ssem.at[b]).wait()

    for b in range(NBUF):
        pltpu.async_copy(p_hbm.at[src_v.at[b]], gbuf.at[b], gsem.at[b])

    def _step(i, carry):
        for b in range(NBUF):
            j = i * NBUF + b
            _gwait(b)

            @pl.when(i > 0)
            def _():
                _swait(b)

            for g in range(CH // 16):
                wv16 = w_v[pl.ds(pl.multiple_of(j * CH + g * 16, 16), 16)]
                for r in range(16):
                    sbuf[b, g * 16 + r] = gbuf[b, g * 16 + r] * wv16[r]

            pltpu.async_copy(sbuf.at[b], acc_sh.at[dst_v.at[j]],
                             ssem.at[b], add=True)

            @pl.when(i < NSTEP - 1)
            def _():
                pltpu.async_copy(p_hbm.at[src_v.at[j + NBUF]], gbuf.at[b],
                                 gsem.at[b])
        return carry
    lax.fori_loop(0, NSTEP, _step, 0)

    for b in range(NBUF):
        _swait(b)
    plsc.subcore_barrier()

    pltpu.sync_copy(acc_sh.at[pl.ds(s * RPT, RPT)], stage_v)
    pltpu.sync_copy(stage_v, out_hbm.at[c, pl.ds(s * RPT, RPT)])


_agg = functools.partial(
    pl.kernel,
    out_type=jax.ShapeDtypeStruct((NC, NPAD, H1), jnp.float32),
    mesh=plsc.VectorSubcoreMesh(core_axis_name="c", subcore_axis_name="s"),
    compiler_params=pltpu.CompilerParams(use_tc_tiling_on_sc=False),
    scratch_types=[
        pltpu.VMEM((NCHUNK, CH), jnp.int32),
        pltpu.VMEM((NCHUNK, CH), jnp.int32),
        pltpu.VMEM((EPWP,), jnp.float32),
        pltpu.VMEM((NBUF, CH, H1), jnp.float32),
        pltpu.VMEM((NBUF, CH, H1), jnp.float32),
        pltpu.VMEM((RPT, H1), jnp.float32),
        pltpu.VMEM_SHARED((NPAD, H1), jnp.float32),
        pltpu.SemaphoreType.DMA((NBUF,)),
        pltpu.SemaphoreType.DMA((NBUF,)),
    ],
)(_agg_body)


def _proj_body(x_ref, wt_ref, o_ref):
    o_ref[...] = jnp.dot(x_ref[...], wt_ref[...],
                         preferred_element_type=jnp.float32,
                         precision=jax.lax.Precision.HIGHEST)


def _combine_relu_body(p_ref, parts_ref, sw_ref, b_ref, o_ref):
    acc = parts_ref[0] + parts_ref[1]
    o_ref[...] = jnp.maximum(
        p_ref[...] * sw_ref[...] + acc + b_ref[...], 0.0)


def _combine_mm_body(x_ref, parts_ref, sw_ref, wt_ref, b_ref, o_ref):
    h2 = x_ref[...] * sw_ref[...] + parts_ref[0] + parts_ref[1]
    o_ref[...] = jnp.dot(h2, wt_ref[...],
                         preferred_element_type=jnp.float32,
                         precision=jax.lax.Precision.HIGHEST) + b_ref[...]


def kernel(features, edge_index, edge_weight, self_weight, W1, b1, W2, b2):
    npad_e = EPWP - EPW
    src = jnp.concatenate(
        [edge_index[0].reshape(NW, EPW),
         jnp.zeros((NW, npad_e), jnp.int32)], axis=1).reshape(NW, NCHUNK, CH)
    dst = jnp.concatenate(
        [edge_index[1].reshape(NW, EPW),
         jnp.full((NW, npad_e), PAD_DST, jnp.int32)],
        axis=1).reshape(NW, NCHUNK, CH)
    w = jnp.concatenate(
        [edge_weight.reshape(NW, EPW),
         jnp.full((NW, npad_e), -1.0, jnp.float32)], axis=1)
    swp1 = self_weight + 1.0

    p1 = pl.pallas_call(
        _proj_body,
        out_shape=jax.ShapeDtypeStruct((N, H1), jnp.float32),
    )(features, W1.T)

    parts1 = _agg(p1, src, dst, w)[:, :N, :]

    x = pl.pallas_call(
        _combine_relu_body,
        out_shape=jax.ShapeDtypeStruct((N, H1), jnp.float32),
    )(p1, parts1, swp1, b1.reshape(1, H1))

    parts2 = _agg(x, src, dst, w)[:, :N, :]

    out = pl.pallas_call(
        _combine_mm_body,
        out_shape=jax.ShapeDtypeStruct((N, W2.shape[0]), jnp.float32),
    )(x, parts2, swp1, W2.T, b2.reshape(1, W2.shape[0]))

    return out


# speedup vs baseline: 14.0235x; 1.0429x over previous
import functools

import jax
import jax.numpy as jnp
from jax import lax
from jax.experimental import pallas as pl
from jax.experimental.pallas import tpu as pltpu, tpu_sc as plsc

N = 10000
E = 320000
D = 128
H1 = 16

NC = 2
NS = 16
NW = NC * NS
EPW = E // NW
CH = 256
EPWP = 10240
NCHUNK = EPWP // CH
NBUF = 4
NSTEP = NCHUNK // NBUF
NPAD = 10240
RPT = NPAD // NS
PAD_DST = NPAD - 8


def _agg_body(p_hbm, src_hbm, dst_hbm, w_hbm, out_hbm,
              src_v, dst_v, w_v, gbuf, sbuf, stage_v, acc_sh, gsem, ssem):
    c = lax.axis_index("c")
    s = lax.axis_index("s")
    wid = c * NS + s

    pltpu.sync_copy(src_hbm.at[wid], src_v)
    pltpu.sync_copy(dst_hbm.at[wid], dst_v)
    pltpu.sync_copy(w_hbm.at[wid], w_v)

    def _wplus(i, carry):
        w_v[pl.ds(i * 16, 16)] = w_v[pl.ds(i * 16, 16)] + 1.0
        return carry
    lax.fori_loop(0, EPWP // 16, _wplus, 0)

    def _zero(i, carry):
        stage_v[i] = jnp.zeros((16,), jnp.float32)
        return carry
    lax.fori_loop(0, RPT, _zero, 0)
    pltpu.sync_copy(stage_v, acc_sh.at[pl.ds(s * RPT, RPT)])
    plsc.subcore_barrier()

    def _gwait(b):
        pltpu.make_async_copy(p_hbm.at[pl.ds(0, CH)], gbuf.at[b],
                              gsem.at[b]).wait()

    def _swait(b):
        pltpu.make_async_copy(p_hbm.at[pl.ds(0, CH)], sbuf.at[b],
                              ssem.at[b]).wait()

    for b in range(NBUF):
        pltpu.async_copy(p_hbm.at[src_v.at[b]], gbuf.at[b], gsem.at[b])

    def _step(i, carry):
        for b in range(NBUF):
            j = i * NBUF + b
            _gwait(b)

            @pl.when(i > 0)
            def _():
                _swait(b)

            def _grp(g, carry2):
                wv16 = w_v[pl.ds(pl.multiple_of(j * CH + g * 16, 16), 16)]
                base = g * 16
                for r in range(16):
                    sbuf[b, base + r] = gbuf[b, base + r] * wv16[r]
                return carry2
            lax.fori_loop(0, CH // 16, _grp, 0)

            pltpu.async_copy(sbuf.at[b], acc_sh.at[dst_v.at[j]],
                             ssem.at[b], add=True)

            @pl.when(i < NSTEP - 1)
            def _():
                pltpu.async_copy(p_hbm.at[src_v.at[j + NBUF]], gbuf.at[b],
                                 gsem.at[b])
        return carry
    lax.fori_loop(0, NSTEP, _step, 0)

    for b in range(NBUF):
        _swait(b)
    plsc.subcore_barrier()

    pltpu.sync_copy(acc_sh.at[pl.ds(s * RPT, RPT)], stage_v)
    pltpu.sync_copy(stage_v, out_hbm.at[c, pl.ds(s * RPT, RPT)])


_agg = functools.partial(
    pl.kernel,
    out_type=jax.ShapeDtypeStruct((NC, NPAD, H1), jnp.float32),
    mesh=plsc.VectorSubcoreMesh(core_axis_name="c", subcore_axis_name="s"),
    compiler_params=pltpu.CompilerParams(use_tc_tiling_on_sc=False),
    scratch_types=[
        pltpu.VMEM((NCHUNK, CH), jnp.int32),
        pltpu.VMEM((NCHUNK, CH), jnp.int32),
        pltpu.VMEM((EPWP,), jnp.float32),
        pltpu.VMEM((NBUF, CH, H1), jnp.float32),
        pltpu.VMEM((NBUF, CH, H1), jnp.float32),
        pltpu.VMEM((RPT, H1), jnp.float32),
        pltpu.VMEM_SHARED((NPAD, H1), jnp.float32),
        pltpu.SemaphoreType.DMA((NBUF,)),
        pltpu.SemaphoreType.DMA((NBUF,)),
    ],
)(_agg_body)


def _proj_body(x_ref, wt_ref, o_ref):
    o_ref[...] = jnp.dot(x_ref[...], wt_ref[...],
                         preferred_element_type=jnp.float32,
                         precision=jax.lax.Precision.HIGHEST)


def _combine_relu_body(p_ref, parts_ref, sw_ref, b_ref, o_ref):
    acc = parts_ref[0] + parts_ref[1]
    o_ref[...] = jnp.maximum(
        p_ref[...] * sw_ref[...] + acc + b_ref[...], 0.0)


def _combine_mm_body(x_ref, parts_ref, sw_ref, wt_ref, b_ref, o_ref):
    h2 = x_ref[...] * sw_ref[...] + parts_ref[0] + parts_ref[1]
    o_ref[...] = jnp.dot(h2, wt_ref[...],
                         preferred_element_type=jnp.float32,
                         precision=jax.lax.Precision.HIGHEST) + b_ref[...]


def _prep_edges(edge_index, edge_weight):
    npad_e = EPWP - EPW
    src = jnp.concatenate(
        [edge_index[0].reshape(NW, EPW),
         jnp.zeros((NW, npad_e), jnp.int32)], axis=1).reshape(NW, NCHUNK, CH)
    dst = jnp.concatenate(
        [edge_index[1].reshape(NW, EPW),
         jnp.full((NW, npad_e), PAD_DST, jnp.int32)],
        axis=1).reshape(NW, NCHUNK, CH)
    w = jnp.concatenate(
        [edge_weight.reshape(NW, EPW),
         jnp.full((NW, npad_e), -1.0, jnp.float32)], axis=1)
    return src, dst, w


def kernel(features, edge_index, edge_weight, self_weight, W1, b1, W2, b2):
    src, dst, w = _prep_edges(edge_index, edge_weight)
    swp1 = self_weight + 1.0

    p1 = pl.pallas_call(
        _proj_body,
        out_shape=jax.ShapeDtypeStruct((N, H1), jnp.float32),
    )(features, W1.T)

    parts1 = _agg(p1, src, dst, w)[:, :N, :]

    x = pl.pallas_call(
        _combine_relu_body,
        out_shape=jax.ShapeDtypeStruct((N, H1), jnp.float32),
    )(p1, parts1, swp1, b1.reshape(1, H1))

    parts2 = _agg(x, src, dst, w)[:, :N, :]

    out = pl.pallas_call(
        _combine_mm_body,
        out_shape=jax.ShapeDtypeStruct((N, W2.shape[0]), jnp.float32),
    )(x, parts2, swp1, W2.T, b2.reshape(1, W2.shape[0]))

    return out
